# baseline (speedup 1.0000x reference)
import functools

import jax
import jax.numpy as jnp
from jax import lax
from jax.experimental import pallas as pl
from jax.experimental.pallas import tpu as pltpu


def _round_up(v, m):
    return (v + m - 1) // m * m


def _gather_kernel(idx_ref,
                   src_ref,
                   out_ref,
                   *, C, blk):
    lane = lax.broadcasted_iota(jnp.int32, (1, 128), 1)
    ones_col = (lane == C).astype(jnp.float32)
    feat_mask = lane < C

    def body(k, _):
        r = idx_ref[k]
        a = src_ref[pl.ds(lax.shift_right_logical(r, 1), 1), :]
        rolled = pltpu.roll(a, 64, axis=1)
        picked = jnp.where((r & 1) == 1, rolled, a)
        out_ref[pl.ds(k, 1), 0, :] = jnp.where(feat_mask, picked, ones_col)
        return 0

    lax.fori_loop(0, blk, body, 0)


def _permute_rows(pillar_features, order, P_pad, C):
    P = pillar_features.shape[0]
    src = pillar_features.astype(jnp.float32).reshape(P // 2, 128)
    idx = jnp.zeros((P_pad,), jnp.int32).at[:P].set(order.astype(jnp.int32))
    BLK = 2048
    n_blk = P_pad // BLK
    out = pl.pallas_call(
        functools.partial(_gather_kernel, C=C, blk=BLK),
        out_shape=jax.ShapeDtypeStruct((P_pad, 1, 128), jnp.float32),
        grid_spec=pltpu.PrefetchScalarGridSpec(
            num_scalar_prefetch=0,
            grid=(2, n_blk // 2),
            in_specs=[
                pl.BlockSpec((BLK,), lambda h, i: (h * (n_blk // 2) + i,),
                             memory_space=pltpu.SMEM),
                pl.BlockSpec((P // 2, 128), lambda h, i: (0, 0)),
            ],
            out_specs=pl.BlockSpec(
                (BLK, 1, 128), lambda h, i: (h * (n_blk // 2) + i, 0, 0)),
        ),
        compiler_params=pltpu.CompilerParams(
            dimension_semantics=("parallel", "arbitrary"),
            vmem_limit_bytes=64 << 20,
        ),
    )(idx, src)
    return out.reshape(P_pad, 128)


def _scatter_kernel(tile_ref, blk_ref, first_ref, last_ref, active_ref,
                    key_ref,
                    pf_ref,
                    out_ref,
                    acc_ref,
                    *, C, tile_s):
    h = pl.program_id(0)
    i = pl.program_id(1)

    @pl.when(first_ref[h, i] == 1)
    def _():
        acc_ref[...] = jnp.zeros_like(acc_ref)

    @pl.when(active_ref[h, i] == 1)
    def _():
        tile_base = tile_ref[h, i] * tile_s
        local = key_ref[0] - tile_base
        pos = lax.broadcasted_iota(jnp.int32, (tile_s, local.shape[1]), 0)
        oh_t = (pos == local).astype(jnp.float32)
        acc_ref[...] += lax.dot_general(
            pf_ref[...], oh_t,
            dimension_numbers=(((0,), (1,)), ((), ())),
            preferred_element_type=jnp.float32)

    @pl.when(last_ref[h, i] == 1)
    def _():
        acc = acc_ref[...]
        counts = acc[C:C + 1, :]
        inv = pl.reciprocal(jnp.maximum(counts, 1.0), approx=False)
        out_ref[...] = (acc[:C, :] * inv)[None].astype(out_ref.dtype)


def _scatter_mean(pillar_features, coords, *, batch_size, nz, ny, nx,
                  tile_s=1024, window=1024):
    P, C = pillar_features.shape
    S = nz * ny * nx
    out_dtype = pillar_features.dtype

    tile_s = _round_up(tile_s, 128)
    S_pad = _round_up(S, tile_s)
    n_s_tiles = S_pad // tile_s
    num_tiles = batch_size * n_s_tiles

    W = _round_up(window, 128)
    P_pad = _round_up(max(P, 1), 4096)
    n_blocks = P_pad // W

    cb = coords[:, 0].astype(jnp.int32)
    cz = coords[:, 1].astype(jnp.int32)
    cy = coords[:, 2].astype(jnp.int32)
    cx = coords[:, 3].astype(jnp.int32)
    flat = cz * (ny * nx) + cy * nx + cx
    valid = ((cb >= 0) & (cb < batch_size) & (cz >= 0) & (cz < nz)
             & (cy >= 0) & (cy < ny) & (cx >= 0) & (cx < nx))
    sentinel = jnp.int32(batch_size * S_pad)
    key = jnp.where(valid, cb * S_pad + flat, sentinel).astype(jnp.int32)

    order = jnp.argsort(key)
    key_pad = jnp.full((P_pad,), sentinel, jnp.int32).at[:P].set(key[order])
    key_row = key_pad.reshape(n_blocks, 1, W)

    pf = _permute_rows(pillar_features, order, P_pad, C)

    bounds = jnp.arange(num_tiles + 1, dtype=jnp.int32) * tile_s
    off = jnp.searchsorted(key_pad, bounds, side="left").astype(jnp.int32)
    seg_len = off[1:] - off[:-1]
    first_blk = jnp.minimum(off[:-1] // W, n_blocks - 1).astype(jnp.int32)
    last_blk = jnp.minimum(jnp.maximum(off[1:] - 1, off[:-1]) // W,
                           n_blocks - 1)
    nblk = jnp.where(seg_len > 0, last_blk - first_blk + 1, 0).astype(jnp.int32)

    T2 = num_tiles // 2
    n_step = T2 + n_blocks
    halves = []
    for hh in range(2):
        nb_h = nblk[hh * T2:(hh + 1) * T2]
        fb_h = first_blk[hh * T2:(hh + 1) * T2]
        nsteps = jnp.maximum(nb_h, 1)
        cum = jnp.concatenate([jnp.zeros((1,), jnp.int32),
                               jnp.cumsum(nsteps).astype(jnp.int32)])
        ii = jnp.arange(n_step, dtype=jnp.int32)
        tloc = jnp.clip(jnp.searchsorted(cum, ii, side="right").astype(jnp.int32) - 1,
                        0, T2 - 1)
        in_range = ii < cum[T2]
        st = hh * T2 + tloc
        j = ii - cum[tloc]
        sb = jnp.clip(fb_h[tloc] + j, 0, n_blocks - 1)
        sf = (in_range & (j == 0)).astype(jnp.int32)
        sl = (in_range & (ii == cum[tloc + 1] - 1)).astype(jnp.int32)
        sa = (in_range & (j < nb_h[tloc])).astype(jnp.int32)
        halves.append((st, sb, sf, sl, sa))
    step_tile, step_blk, step_first, step_last, step_active = (
        jnp.stack([h[k] for h in halves]) for k in range(5))

    _body = functools.partial(_scatter_kernel, C=C, tile_s=tile_s)

    out = pl.pallas_call(
        _body,
        out_shape=jax.ShapeDtypeStruct((batch_size, C, S_pad), out_dtype),
        grid_spec=pltpu.PrefetchScalarGridSpec(
            num_scalar_prefetch=5,
            grid=(2, n_step),
            in_specs=[
                pl.BlockSpec((1, 1, W),
                             lambda h, i, st, sb, *_: (sb[h, i], 0, 0)),
                pl.BlockSpec((W, 128),
                             lambda h, i, st, sb, *_: (sb[h, i], 0)),
            ],
            out_specs=pl.BlockSpec(
                (1, C, tile_s),
                lambda h, i, st, sb, *_: (st[h, i] // n_s_tiles, 0,
                                          st[h, i] % n_s_tiles)),
            scratch_shapes=[pltpu.VMEM((128, tile_s), jnp.float32)],
        ),
        compiler_params=pltpu.CompilerParams(
            dimension_semantics=("parallel", "arbitrary"),
            vmem_limit_bytes=100 << 20,
        ),
    )(step_tile, step_blk, step_first, step_last, step_active, key_row, pf)

    if S_pad != S:
        out = out[:, :, :S]
    return out.reshape(batch_size, C * nz, ny, nx)


def kernel(pillar_features, coords):
    return _scatter_mean(pillar_features, coords,
                         batch_size=4, nz=2, ny=256, nx=256)


# speedup vs baseline: 4.0335x; 4.0335x over previous
import functools

import jax
import jax.numpy as jnp
from jax import lax
from jax.experimental import pallas as pl
from jax.experimental.pallas import tpu as pltpu


def _round_up(v, m):
    return (v + m - 1) // m * m


def _gather_kernel(idx_ref,
                   src_ref,
                   out_ref,
                   *, C, blk):
    lane = lax.broadcasted_iota(jnp.int32, (1, 128), 1)
    one_at_c = lane == C
    one_f = jnp.ones((1, 128), jnp.float32)
    U = 8

    def body(g, _):
        base = g * U
        for u in range(U):
            k = base + u
            r = idx_ref[k]
            a = src_ref[pl.ds(lax.shift_right_logical(r, 1), 1), :]
            rolled = pltpu.roll(a, 64, axis=1)
            picked = jnp.where((r & 1) == 1, rolled, a)
            out_ref[pl.ds(k, 1), 0, :] = jnp.where(one_at_c, one_f, picked)
        return 0

    lax.fori_loop(0, blk // U, body, 0)


def _permute_rows(pillar_features, order, P_pad, C):
    P = pillar_features.shape[0]
    src = pillar_features.astype(jnp.float32).reshape(P // 2, 128)
    idx = jnp.zeros((P_pad,), jnp.int32).at[:P].set(order.astype(jnp.int32))
    BLK = 2048
    n_blk = P_pad // BLK
    out = pl.pallas_call(
        functools.partial(_gather_kernel, C=C, blk=BLK),
        out_shape=jax.ShapeDtypeStruct((P_pad, 1, 128), jnp.float32),
        grid_spec=pltpu.PrefetchScalarGridSpec(
            num_scalar_prefetch=0,
            grid=(2, n_blk // 2),
            in_specs=[
                pl.BlockSpec((BLK,), lambda h, i: (h * (n_blk // 2) + i,),
                             memory_space=pltpu.SMEM),
                pl.BlockSpec((P // 2, 128), lambda h, i: (0, 0)),
            ],
            out_specs=pl.BlockSpec(
                (BLK, 1, 128), lambda h, i: (h * (n_blk // 2) + i, 0, 0)),
        ),
        compiler_params=pltpu.CompilerParams(
            dimension_semantics=("parallel", "arbitrary"),
            vmem_limit_bytes=64 << 20,
        ),
    )(idx, src)
    return out.reshape(P_pad, 128)


def _scatter_kernel(tile_ref, blk_ref, first_ref, last_ref, active_ref,
                    key_ref,
                    pf_ref,
                    out_ref,
                    acc_ref,
                    *, C, tile_s):
    h = pl.program_id(0)
    i = pl.program_id(1)

    @pl.when(first_ref[h, i] == 1)
    def _():
        acc_ref[...] = jnp.zeros_like(acc_ref)

    @pl.when(active_ref[h, i] == 1)
    def _():
        tile_base = tile_ref[h, i] * tile_s
        local = key_ref[0] - tile_base
        pos = lax.broadcasted_iota(jnp.int32, (tile_s, local.shape[1]), 0)
        oh_t = (pos == local).astype(jnp.float32)
        acc_ref[...] += lax.dot_general(
            pf_ref[...], oh_t,
            dimension_numbers=(((0,), (1,)), ((), ())),
            preferred_element_type=jnp.float32)

    @pl.when(last_ref[h, i] == 1)
    def _():
        acc = acc_ref[...]
        counts = acc[C:C + 1, :]
        inv = pl.reciprocal(jnp.maximum(counts, 1.0), approx=False)
        out_ref[...] = (acc[:C, :] * inv)[None].astype(out_ref.dtype)


def _scatter_mean(pillar_features, coords, *, batch_size, nz, ny, nx,
                  tile_s=1024, window=1024):
    P, C = pillar_features.shape
    S = nz * ny * nx
    out_dtype = pillar_features.dtype

    tile_s = _round_up(tile_s, 128)
    S_pad = _round_up(S, tile_s)
    n_s_tiles = S_pad // tile_s
    num_tiles = batch_size * n_s_tiles

    W = _round_up(window, 128)
    P_pad = _round_up(max(P, 1), 4096)
    n_blocks = P_pad // W

    cb = coords[:, 0].astype(jnp.int32)
    cz = coords[:, 1].astype(jnp.int32)
    cy = coords[:, 2].astype(jnp.int32)
    cx = coords[:, 3].astype(jnp.int32)
    flat = cz * (ny * nx) + cy * nx + cx
    valid = ((cb >= 0) & (cb < batch_size) & (cz >= 0) & (cz < nz)
             & (cy >= 0) & (cy < ny) & (cx >= 0) & (cx < nx))
    sentinel = jnp.int32(batch_size * S_pad)
    key = jnp.where(valid, cb * S_pad + flat, sentinel).astype(jnp.int32)

    order = jnp.argsort(key)
    key_pad = jnp.full((P_pad,), sentinel, jnp.int32).at[:P].set(key[order])
    key_row = key_pad.reshape(n_blocks, 1, W)

    pf = _permute_rows(pillar_features, order, P_pad, C)

    bounds = jnp.arange(num_tiles + 1, dtype=jnp.int32) * tile_s
    off = jnp.searchsorted(key_pad, bounds, side="left").astype(jnp.int32)
    seg_len = off[1:] - off[:-1]
    first_blk = jnp.minimum(off[:-1] // W, n_blocks - 1).astype(jnp.int32)
    last_blk = jnp.minimum(jnp.maximum(off[1:] - 1, off[:-1]) // W,
                           n_blocks - 1)
    nblk = jnp.where(seg_len > 0, last_blk - first_blk + 1, 0).astype(jnp.int32)

    T2 = num_tiles // 2
    n_step = T2 + n_blocks
    halves = []
    for hh in range(2):
        nb_h = nblk[hh * T2:(hh + 1) * T2]
        fb_h = first_blk[hh * T2:(hh + 1) * T2]
        nsteps = jnp.maximum(nb_h, 1)
        cum = jnp.concatenate([jnp.zeros((1,), jnp.int32),
                               jnp.cumsum(nsteps).astype(jnp.int32)])
        ii = jnp.arange(n_step, dtype=jnp.int32)
        tloc = jnp.clip(jnp.searchsorted(cum, ii, side="right").astype(jnp.int32) - 1,
                        0, T2 - 1)
        in_range = ii < cum[T2]
        st = hh * T2 + tloc
        j = ii - cum[tloc]
        sb = jnp.clip(fb_h[tloc] + j, 0, n_blocks - 1)
        sf = (in_range & (j == 0)).astype(jnp.int32)
        sl = (in_range & (ii == cum[tloc + 1] - 1)).astype(jnp.int32)
        sa = (in_range & (j < nb_h[tloc])).astype(jnp.int32)
        halves.append((st, sb, sf, sl, sa))
    step_tile, step_blk, step_first, step_last, step_active = (
        jnp.stack([h[k] for h in halves]) for k in range(5))

    _body = functools.partial(_scatter_kernel, C=C, tile_s=tile_s)

    out = pl.pallas_call(
        _body,
        out_shape=jax.ShapeDtypeStruct((batch_size, C, S_pad), out_dtype),
        grid_spec=pltpu.PrefetchScalarGridSpec(
            num_scalar_prefetch=5,
            grid=(2, n_step),
            in_specs=[
                pl.BlockSpec((1, 1, W),
                             lambda h, i, st, sb, *_: (sb[h, i], 0, 0)),
                pl.BlockSpec((W, 128),
                             lambda h, i, st, sb, *_: (sb[h, i], 0)),
            ],
            out_specs=pl.BlockSpec(
                (1, C, tile_s),
                lambda h, i, st, sb, *_: (st[h, i] // n_s_tiles, 0,
                                          st[h, i] % n_s_tiles)),
            scratch_shapes=[pltpu.VMEM((128, tile_s), jnp.float32)],
        ),
        compiler_params=pltpu.CompilerParams(
            dimension_semantics=("parallel", "arbitrary"),
            vmem_limit_bytes=100 << 20,
        ),
    )(step_tile, step_blk, step_first, step_last, step_active, key_row, pf)

    if S_pad != S:
        out = out[:, :, :S]
    return out.reshape(batch_size, C * nz, ny, nx)


def kernel(pillar_features, coords):
    return _scatter_mean(pillar_features, coords,
                         batch_size=4, nz=2, ny=256, nx=256)


# speedup vs baseline: 5.0974x; 1.2638x over previous
import functools

import jax
import jax.numpy as jnp
from jax import lax
from jax.experimental import pallas as pl
from jax.experimental.pallas import tpu as pltpu


def _round_up(v, m):
    return (v + m - 1) // m * m


def _gather_kernel(idx_ref,
                   src_ref,
                   out_ref,
                   *, C, blk):
    lane = lax.broadcasted_iota(jnp.int32, (1, 128), 1)
    one_at_c = lane == C
    one_f = jnp.ones((1, 128), jnp.float32)
    U = 16

    def body(g, _):
        base = g * U
        for u in range(U):
            k = base + u
            r = idx_ref[k]
            a = src_ref[pl.ds(lax.shift_right_logical(r, 1), 1), 0, :]
            rolled = pltpu.roll(a, 64, axis=1)
            picked = jnp.where((r & 1) == 1, rolled, a)
            out_ref[pl.ds(k, 1), 0, :] = jnp.where(one_at_c, one_f, picked)
        return 0

    lax.fori_loop(0, blk // U, body, 0)


def _permute_rows(pillar_features, order, P_pad, C):
    P = pillar_features.shape[0]
    src = pillar_features.astype(jnp.float32).reshape(P // 2, 1, 128)
    idx = jnp.zeros((P_pad,), jnp.int32).at[:P].set(order.astype(jnp.int32))
    BLK = 2048
    n_blk = P_pad // BLK
    out = pl.pallas_call(
        functools.partial(_gather_kernel, C=C, blk=BLK),
        out_shape=jax.ShapeDtypeStruct((P_pad, 1, 128), jnp.float32),
        grid_spec=pltpu.PrefetchScalarGridSpec(
            num_scalar_prefetch=0,
            grid=(2, n_blk // 2),
            in_specs=[
                pl.BlockSpec((BLK,), lambda h, i: (h * (n_blk // 2) + i,),
                             memory_space=pltpu.SMEM),
                pl.BlockSpec((P // 2, 1, 128), lambda h, i: (0, 0, 0)),
            ],
            out_specs=pl.BlockSpec(
                (BLK, 1, 128), lambda h, i: (h * (n_blk // 2) + i, 0, 0)),
        ),
        compiler_params=pltpu.CompilerParams(
            dimension_semantics=("parallel", "arbitrary"),
            vmem_limit_bytes=64 << 20,
        ),
    )(idx, src)
    return out.reshape(P_pad, 128)


def _scatter_kernel(tile_ref, blk_ref, first_ref, last_ref, active_ref,
                    key_ref,
                    pf_ref,
                    out_ref,
                    acc_ref,
                    *, C, tile_s):
    h = pl.program_id(0)
    i = pl.program_id(1)

    @pl.when(first_ref[h, i] == 1)
    def _():
        acc_ref[...] = jnp.zeros_like(acc_ref)

    @pl.when(active_ref[h, i] == 1)
    def _():
        tile_base = tile_ref[h, i] * tile_s
        local = key_ref[0] - tile_base
        pos = lax.broadcasted_iota(jnp.int32, (tile_s, local.shape[1]), 0)
        oh_t = (pos == local).astype(jnp.float32)
        acc_ref[...] += lax.dot_general(
            pf_ref[...], oh_t,
            dimension_numbers=(((0,), (1,)), ((), ())),
            preferred_element_type=jnp.float32)

    @pl.when(last_ref[h, i] == 1)
    def _():
        acc = acc_ref[...]
        counts = acc[C:C + 1, :]
        inv = pl.reciprocal(jnp.maximum(counts, 1.0), approx=False)
        out_ref[...] = (acc[:C, :] * inv)[None].astype(out_ref.dtype)


def _scatter_mean(pillar_features, coords, *, batch_size, nz, ny, nx,
                  tile_s=1024, window=1024):
    P, C = pillar_features.shape
    S = nz * ny * nx
    out_dtype = pillar_features.dtype

    tile_s = _round_up(tile_s, 128)
    S_pad = _round_up(S, tile_s)
    n_s_tiles = S_pad // tile_s
    num_tiles = batch_size * n_s_tiles

    W = _round_up(window, 128)
    P_pad = _round_up(max(P, 1), 4096)
    n_blocks = P_pad // W

    cb = coords[:, 0].astype(jnp.int32)
    cz = coords[:, 1].astype(jnp.int32)
    cy = coords[:, 2].astype(jnp.int32)
    cx = coords[:, 3].astype(jnp.int32)
    flat = cz * (ny * nx) + cy * nx + cx
    valid = ((cb >= 0) & (cb < batch_size) & (cz >= 0) & (cz < nz)
             & (cy >= 0) & (cy < ny) & (cx >= 0) & (cx < nx))
    sentinel = jnp.int32(batch_size * S_pad)
    key = jnp.where(valid, cb * S_pad + flat, sentinel).astype(jnp.int32)

    order = jnp.argsort(key)
    key_pad = jnp.full((P_pad,), sentinel, jnp.int32).at[:P].set(key[order])
    key_row = key_pad.reshape(n_blocks, 1, W)

    pf = _permute_rows(pillar_features, order, P_pad, C)

    bounds = jnp.arange(num_tiles + 1, dtype=jnp.int32) * tile_s
    off = jnp.searchsorted(key_pad, bounds, side="left").astype(jnp.int32)
    seg_len = off[1:] - off[:-1]
    first_blk = jnp.minimum(off[:-1] // W, n_blocks - 1).astype(jnp.int32)
    last_blk = jnp.minimum(jnp.maximum(off[1:] - 1, off[:-1]) // W,
                           n_blocks - 1)
    nblk = jnp.where(seg_len > 0, last_blk - first_blk + 1, 0).astype(jnp.int32)

    T2 = num_tiles // 2
    n_step = T2 + n_blocks
    halves = []
    for hh in range(2):
        nb_h = nblk[hh * T2:(hh + 1) * T2]
        fb_h = first_blk[hh * T2:(hh + 1) * T2]
        nsteps = jnp.maximum(nb_h, 1)
        cum = jnp.concatenate([jnp.zeros((1,), jnp.int32),
                               jnp.cumsum(nsteps).astype(jnp.int32)])
        ii = jnp.arange(n_step, dtype=jnp.int32)
        tloc = jnp.clip(jnp.searchsorted(cum, ii, side="right").astype(jnp.int32) - 1,
                        0, T2 - 1)
        in_range = ii < cum[T2]
        st = hh * T2 + tloc
        j = ii - cum[tloc]
        sb = jnp.clip(fb_h[tloc] + j, 0, n_blocks - 1)
        sf = (in_range & (j == 0)).astype(jnp.int32)
        sl = (in_range & (ii == cum[tloc + 1] - 1)).astype(jnp.int32)
        sa = (in_range & (j < nb_h[tloc])).astype(jnp.int32)
        halves.append((st, sb, sf, sl, sa))
    step_tile, step_blk, step_first, step_last, step_active = (
        jnp.stack([h[k] for h in halves]) for k in range(5))

    _body = functools.partial(_scatter_kernel, C=C, tile_s=tile_s)

    out = pl.pallas_call(
        _body,
        out_shape=jax.ShapeDtypeStruct((batch_size, C, S_pad), out_dtype),
        grid_spec=pltpu.PrefetchScalarGridSpec(
            num_scalar_prefetch=5,
            grid=(2, n_step),
            in_specs=[
                pl.BlockSpec((1, 1, W),
                             lambda h, i, st, sb, *_: (sb[h, i], 0, 0)),
                pl.BlockSpec((W, 128),
                             lambda h, i, st, sb, *_: (sb[h, i], 0)),
            ],
            out_specs=pl.BlockSpec(
                (1, C, tile_s),
                lambda h, i, st, sb, *_: (st[h, i] // n_s_tiles, 0,
                                          st[h, i] % n_s_tiles)),
            scratch_shapes=[pltpu.VMEM((128, tile_s), jnp.float32)],
        ),
        compiler_params=pltpu.CompilerParams(
            dimension_semantics=("parallel", "arbitrary"),
            vmem_limit_bytes=100 << 20,
        ),
    )(step_tile, step_blk, step_first, step_last, step_active, key_row, pf)

    if S_pad != S:
        out = out[:, :, :S]
    return out.reshape(batch_size, C * nz, ny, nx)


def kernel(pillar_features, coords):
    return _scatter_mean(pillar_features, coords,
                         batch_size=4, nz=2, ny=256, nx=256)


# speedup vs baseline: 5.2211x; 1.0243x over previous
import functools

import jax
import jax.numpy as jnp
from jax import lax
from jax.experimental import pallas as pl
from jax.experimental.pallas import tpu as pltpu


def _round_up(v, m):
    return (v + m - 1) // m * m


def _gather_kernel(idx_ref,
                   src_ref,
                   out_ref,
                   *, C, blk):
    lane = lax.broadcasted_iota(jnp.int32, (1, 128), 1)
    one_at_c = lane == C
    one_f = jnp.ones((1, 128), jnp.float32)
    U = 16

    def body(g, _):
        base = g * U
        for u in range(U):
            k = base + u
            r = idx_ref[k]
            a = src_ref[pl.ds(lax.shift_right_logical(r, 1), 1), 0, :]
            rolled = pltpu.roll(a, 64, axis=1)
            picked = jnp.where((r & 1) == 1, rolled, a)
            out_ref[pl.ds(k, 1), 0, :] = jnp.where(one_at_c, one_f, picked)
        return 0

    lax.fori_loop(0, blk // U, body, 0)


def _permute_rows(pillar_features, order, P_pad, C):
    P = pillar_features.shape[0]
    src = pillar_features.astype(jnp.float32).reshape(P // 2, 1, 128)
    idx = jnp.zeros((P_pad,), jnp.int32).at[:P].set(order.astype(jnp.int32))
    BLK = 2048
    n_blk = P_pad // BLK
    out = pl.pallas_call(
        functools.partial(_gather_kernel, C=C, blk=BLK),
        out_shape=jax.ShapeDtypeStruct((P_pad, 1, 128), jnp.float32),
        grid_spec=pltpu.PrefetchScalarGridSpec(
            num_scalar_prefetch=0,
            grid=(2, n_blk // 2),
            in_specs=[
                pl.BlockSpec((BLK,), lambda h, i: (h * (n_blk // 2) + i,),
                             memory_space=pltpu.SMEM),
                pl.BlockSpec((P // 2, 1, 128), lambda h, i: (0, 0, 0)),
            ],
            out_specs=pl.BlockSpec(
                (BLK, 1, 128), lambda h, i: (h * (n_blk // 2) + i, 0, 0)),
        ),
        compiler_params=pltpu.CompilerParams(
            dimension_semantics=("parallel", "arbitrary"),
            vmem_limit_bytes=64 << 20,
        ),
    )(idx, src)
    return out.reshape(P_pad, 128)


def _scatter_kernel(tile_ref, blk_ref, first_ref, last_ref, active_ref,
                    key_ref,
                    pf_ref,
                    out_ref,
                    acc_ref,
                    *, C, tile_s):
    h = pl.program_id(0)
    i = pl.program_id(1)

    @pl.when(first_ref[h, i] == 1)
    def _():
        acc_ref[...] = jnp.zeros_like(acc_ref)

    @pl.when(active_ref[h, i] == 1)
    def _():
        tile_base = tile_ref[h, i] * tile_s
        local = key_ref[0] - tile_base
        pos = lax.broadcasted_iota(jnp.int32, (tile_s, local.shape[1]), 0)
        oh_t = (pos == local).astype(jnp.float32)
        acc_ref[...] += lax.dot_general(
            pf_ref[...], oh_t,
            dimension_numbers=(((0,), (1,)), ((), ())),
            preferred_element_type=jnp.float32)

    @pl.when(last_ref[h, i] == 1)
    def _():
        acc = acc_ref[...]
        counts = acc[C:C + 1, :]
        inv = pl.reciprocal(jnp.maximum(counts, 1.0), approx=False)
        out_ref[...] = (acc[:C, :] * inv)[None].astype(out_ref.dtype)


def _scatter_mean(pillar_features, coords, *, batch_size, nz, ny, nx,
                  tile_s=1024, window=1024):
    P, C = pillar_features.shape
    S = nz * ny * nx
    out_dtype = pillar_features.dtype

    tile_s = _round_up(tile_s, 128)
    S_pad = _round_up(S, tile_s)
    n_s_tiles = S_pad // tile_s
    num_tiles = batch_size * n_s_tiles

    W = _round_up(window, 128)
    P_pad = _round_up(max(P, 1), 4096)
    n_blocks = P_pad // W

    cb = coords[:, 0].astype(jnp.int32)
    cz = coords[:, 1].astype(jnp.int32)
    cy = coords[:, 2].astype(jnp.int32)
    cx = coords[:, 3].astype(jnp.int32)
    flat = cz * (ny * nx) + cy * nx + cx
    valid = ((cb >= 0) & (cb < batch_size) & (cz >= 0) & (cz < nz)
             & (cy >= 0) & (cy < ny) & (cx >= 0) & (cx < nx))
    sentinel = jnp.int32(batch_size * S_pad)
    key = jnp.where(valid, cb * S_pad + flat, sentinel).astype(jnp.int32)

    order = jnp.argsort(key)
    key_pad = jnp.full((P_pad,), sentinel, jnp.int32).at[:P].set(key[order])
    key_row = key_pad.reshape(n_blocks, 1, W)

    pf = _permute_rows(pillar_features, order, P_pad, C)

    bounds = jnp.arange(num_tiles + 1, dtype=jnp.int32) * tile_s
    off = jnp.searchsorted(key_pad, bounds, side="left").astype(jnp.int32)
    seg_len = off[1:] - off[:-1]
    first_blk = jnp.minimum(off[:-1] // W, n_blocks - 1).astype(jnp.int32)
    last_blk = jnp.minimum(jnp.maximum(off[1:] - 1, off[:-1]) // W,
                           n_blocks - 1)
    nblk = jnp.where(seg_len > 0, last_blk - first_blk + 1, 0).astype(jnp.int32)

    T2 = num_tiles // 2
    n_step = T2 + n_blocks
    nb2 = nblk.reshape(2, T2)
    fb2 = first_blk.reshape(2, T2)
    nsteps = jnp.maximum(nb2, 1)
    cum = jnp.concatenate([jnp.zeros((2, 1), jnp.int32),
                           jnp.cumsum(nsteps, axis=1).astype(jnp.int32)],
                          axis=1)
    ii = jnp.arange(n_step, dtype=jnp.int32)[None, :]
    tloc = (jnp.sum(cum[:, :, None] <= ii[:, None, :], axis=1)
            .astype(jnp.int32) - 1)
    tloc = jnp.clip(tloc, 0, T2 - 1)
    in_range = ii < cum[:, T2:T2 + 1]
    st = jnp.arange(2, dtype=jnp.int32)[:, None] * T2 + tloc
    cum_t = jnp.take_along_axis(cum, tloc, axis=1)
    j = ii - cum_t
    sb = jnp.clip(jnp.take_along_axis(fb2, tloc, axis=1) + j, 0, n_blocks - 1)
    sf = (in_range & (j == 0)).astype(jnp.int32)
    sl = (in_range & (ii == jnp.take_along_axis(cum, tloc + 1, axis=1) - 1)
          ).astype(jnp.int32)
    sa = (in_range & (j < jnp.take_along_axis(nb2, tloc, axis=1))
          ).astype(jnp.int32)
    step_tile, step_blk, step_first, step_last, step_active = st, sb, sf, sl, sa

    _body = functools.partial(_scatter_kernel, C=C, tile_s=tile_s)

    out = pl.pallas_call(
        _body,
        out_shape=jax.ShapeDtypeStruct((batch_size, C, S_pad), out_dtype),
        grid_spec=pltpu.PrefetchScalarGridSpec(
            num_scalar_prefetch=5,
            grid=(2, n_step),
            in_specs=[
                pl.BlockSpec((1, 1, W),
                             lambda h, i, st, sb, *_: (sb[h, i], 0, 0)),
                pl.BlockSpec((W, 128),
                             lambda h, i, st, sb, *_: (sb[h, i], 0)),
            ],
            out_specs=pl.BlockSpec(
                (1, C, tile_s),
                lambda h, i, st, sb, *_: (st[h, i] // n_s_tiles, 0,
                                          st[h, i] % n_s_tiles)),
            scratch_shapes=[pltpu.VMEM((128, tile_s), jnp.float32)],
        ),
        compiler_params=pltpu.CompilerParams(
            dimension_semantics=("parallel", "arbitrary"),
            vmem_limit_bytes=100 << 20,
        ),
    )(step_tile, step_blk, step_first, step_last, step_active, key_row, pf)

    if S_pad != S:
        out = out[:, :, :S]
    return out.reshape(batch_size, C * nz, ny, nx)


def kernel(pillar_features, coords):
    return _scatter_mean(pillar_features, coords,
                         batch_size=4, nz=2, ny=256, nx=256)


# speedup vs baseline: 5.2476x; 1.0051x over previous
import functools

import jax
import jax.numpy as jnp
from jax import lax
from jax.experimental import pallas as pl
from jax.experimental.pallas import tpu as pltpu


def _round_up(v, m):
    return (v + m - 1) // m * m


def _gather_kernel(idx_ref,
                   src_ref,
                   out_ref,
                   *, C, blk):
    lane = lax.broadcasted_iota(jnp.int32, (1, 128), 1)
    one_at_c = lane == C
    one_f = jnp.ones((1, 128), jnp.float32)
    U = 16

    def body(g, _):
        base = g * U
        for u in range(U):
            k = base + u
            r = idx_ref[k]
            a = src_ref[pl.ds(lax.shift_right_logical(r, 1), 1), 0, :]
            rolled = pltpu.roll(a, 64, axis=1)
            picked = jnp.where((r & 1) == 1, rolled, a)
            out_ref[pl.ds(k, 1), 0, :] = jnp.where(one_at_c, one_f, picked)
        return 0

    lax.fori_loop(0, blk // U, body, 0)


def _permute_rows(pillar_features, order, P_pad, C):
    P = pillar_features.shape[0]
    src = pillar_features.astype(jnp.float32).reshape(P // 2, 1, 128)
    idx = jnp.zeros((P_pad,), jnp.int32).at[:P].set(order.astype(jnp.int32))
    BLK = 2048
    n_blk = P_pad // BLK
    out = pl.pallas_call(
        functools.partial(_gather_kernel, C=C, blk=BLK),
        out_shape=jax.ShapeDtypeStruct((P_pad, 1, 128), jnp.float32),
        grid_spec=pltpu.PrefetchScalarGridSpec(
            num_scalar_prefetch=0,
            grid=(2, n_blk // 2),
            in_specs=[
                pl.BlockSpec((BLK,), lambda h, i: (h * (n_blk // 2) + i,),
                             memory_space=pltpu.SMEM),
                pl.BlockSpec((P // 2, 1, 128), lambda h, i: (0, 0, 0)),
            ],
            out_specs=pl.BlockSpec(
                (BLK, 1, 128), lambda h, i: (h * (n_blk // 2) + i, 0, 0)),
        ),
        compiler_params=pltpu.CompilerParams(
            dimension_semantics=("parallel", "arbitrary"),
            vmem_limit_bytes=64 << 20,
        ),
    )(idx, src)
    return out.reshape(P_pad, 128)


def _scatter_kernel(tile_ref, blk_ref, first_ref, last_ref, active_ref,
                    key_ref,
                    pf_ref,
                    out_ref,
                    acc_ref,
                    *, C, tile_s):
    h = pl.program_id(0)
    i = pl.program_id(1)

    @pl.when(first_ref[h, i] == 1)
    def _():
        acc_ref[...] = jnp.zeros_like(acc_ref)

    @pl.when(active_ref[h, i] == 1)
    def _():
        tile_base = tile_ref[h, i] * tile_s
        local = key_ref[0] - tile_base
        pos = lax.broadcasted_iota(jnp.int32, (tile_s, local.shape[1]), 0)
        oh_t = (pos == local).astype(jnp.float32)
        acc_ref[...] += lax.dot_general(
            pf_ref[...], oh_t,
            dimension_numbers=(((0,), (1,)), ((), ())),
            preferred_element_type=jnp.float32)

    @pl.when(last_ref[h, i] == 1)
    def _():
        acc = acc_ref[...]
        counts = acc[C:C + 1, :]
        inv = pl.reciprocal(jnp.maximum(counts, 1.0), approx=False)
        out_ref[...] = (acc[:C, :] * inv)[None].astype(out_ref.dtype)


def _scatter_mean(pillar_features, coords, *, batch_size, nz, ny, nx,
                  tile_s=1024, window=512):
    P, C = pillar_features.shape
    S = nz * ny * nx
    out_dtype = pillar_features.dtype

    tile_s = _round_up(tile_s, 128)
    S_pad = _round_up(S, tile_s)
    n_s_tiles = S_pad // tile_s
    num_tiles = batch_size * n_s_tiles

    W = _round_up(window, 128)
    P_pad = _round_up(max(P, 1), 4096)
    n_blocks = P_pad // W

    cb = coords[:, 0].astype(jnp.int32)
    cz = coords[:, 1].astype(jnp.int32)
    cy = coords[:, 2].astype(jnp.int32)
    cx = coords[:, 3].astype(jnp.int32)
    flat = cz * (ny * nx) + cy * nx + cx
    valid = ((cb >= 0) & (cb < batch_size) & (cz >= 0) & (cz < nz)
             & (cy >= 0) & (cy < ny) & (cx >= 0) & (cx < nx))
    sentinel = jnp.int32(batch_size * S_pad)
    key = jnp.where(valid, cb * S_pad + flat, sentinel).astype(jnp.int32)

    order = jnp.argsort(key)
    key_pad = jnp.full((P_pad,), sentinel, jnp.int32).at[:P].set(key[order])
    key_row = key_pad.reshape(n_blocks, 1, W)

    pf = _permute_rows(pillar_features, order, P_pad, C)

    bounds = jnp.arange(num_tiles + 1, dtype=jnp.int32) * tile_s
    off = jnp.searchsorted(key_pad, bounds, side="left").astype(jnp.int32)
    seg_len = off[1:] - off[:-1]
    first_blk = jnp.minimum(off[:-1] // W, n_blocks - 1).astype(jnp.int32)
    last_blk = jnp.minimum(jnp.maximum(off[1:] - 1, off[:-1]) // W,
                           n_blocks - 1)
    nblk = jnp.where(seg_len > 0, last_blk - first_blk + 1, 0).astype(jnp.int32)

    T2 = num_tiles // 2
    n_step = T2 + n_blocks
    nb2 = nblk.reshape(2, T2)
    fb2 = first_blk.reshape(2, T2)
    nsteps = jnp.maximum(nb2, 1)
    cum = jnp.concatenate([jnp.zeros((2, 1), jnp.int32),
                           jnp.cumsum(nsteps, axis=1).astype(jnp.int32)],
                          axis=1)
    ii = jnp.arange(n_step, dtype=jnp.int32)[None, :]
    tloc = (jnp.sum(cum[:, :, None] <= ii[:, None, :], axis=1)
            .astype(jnp.int32) - 1)
    tloc = jnp.clip(tloc, 0, T2 - 1)
    in_range = ii < cum[:, T2:T2 + 1]
    st = jnp.arange(2, dtype=jnp.int32)[:, None] * T2 + tloc
    cum_t = jnp.take_along_axis(cum, tloc, axis=1)
    j = ii - cum_t
    sb = jnp.clip(jnp.take_along_axis(fb2, tloc, axis=1) + j, 0, n_blocks - 1)
    sf = (in_range & (j == 0)).astype(jnp.int32)
    sl = (in_range & (ii == jnp.take_along_axis(cum, tloc + 1, axis=1) - 1)
          ).astype(jnp.int32)
    sa = (in_range & (j < jnp.take_along_axis(nb2, tloc, axis=1))
          ).astype(jnp.int32)
    step_tile, step_blk, step_first, step_last, step_active = st, sb, sf, sl, sa

    _body = functools.partial(_scatter_kernel, C=C, tile_s=tile_s)

    out = pl.pallas_call(
        _body,
        out_shape=jax.ShapeDtypeStruct((batch_size, C, S_pad), out_dtype),
        grid_spec=pltpu.PrefetchScalarGridSpec(
            num_scalar_prefetch=5,
            grid=(2, n_step),
            in_specs=[
                pl.BlockSpec((1, 1, W),
                             lambda h, i, st, sb, *_: (sb[h, i], 0, 0)),
                pl.BlockSpec((W, 128),
                             lambda h, i, st, sb, *_: (sb[h, i], 0)),
            ],
            out_specs=pl.BlockSpec(
                (1, C, tile_s),
                lambda h, i, st, sb, *_: (st[h, i] // n_s_tiles, 0,
                                          st[h, i] % n_s_tiles)),
            scratch_shapes=[pltpu.VMEM((128, tile_s), jnp.float32)],
        ),
        compiler_params=pltpu.CompilerParams(
            dimension_semantics=("parallel", "arbitrary"),
            vmem_limit_bytes=100 << 20,
        ),
    )(step_tile, step_blk, step_first, step_last, step_active, key_row, pf)

    if S_pad != S:
        out = out[:, :, :S]
    return out.reshape(batch_size, C * nz, ny, nx)


def kernel(pillar_features, coords):
    return _scatter_mean(pillar_features, coords,
                         batch_size=4, nz=2, ny=256, nx=256)
